# baseline (speedup 1.0000x reference)
import jax
import jax.numpy as jnp
from jax.experimental import pallas as pl
from jax.experimental.pallas import tpu as pltpu

_NCHUNK = 8


def _copy_body(f_in, m_in, f_out, m_out, mask_out, sems, sem_m):
    mask_out[...] = jnp.ones(mask_out.shape, dtype=jnp.bool_)
    rows = f_in.shape[0] // _NCHUNK
    copies = []
    for i in range(_NCHUNK):
        c = pltpu.make_async_copy(
            f_in.at[pl.ds(i * rows, rows), :],
            f_out.at[pl.ds(i * rows, rows), :],
            sems.at[i],
        )
        c.start()
        copies.append(c)
    cm = pltpu.make_async_copy(m_in, m_out, sem_m)
    cm.start()
    for c in copies:
        c.wait()
    cm.wait()


def kernel(features, means, xy_coords, A):
    B, V, G, C = features.shape
    del xy_coords, A
    n = B * V * G * C
    f2 = features.reshape(256, n // 256)
    m2 = means.reshape(B * V, G * 3)

    f_out, m_out, mask = pl.pallas_call(
        _copy_body,
        in_specs=[
            pl.BlockSpec(memory_space=pl.ANY),
            pl.BlockSpec(memory_space=pl.ANY),
        ],
        out_specs=[
            pl.BlockSpec(memory_space=pl.ANY),
            pl.BlockSpec(memory_space=pl.ANY),
            pl.BlockSpec(memory_space=pltpu.MemorySpace.VMEM),
        ],
        out_shape=[
            jax.ShapeDtypeStruct((256, n // 256), features.dtype),
            jax.ShapeDtypeStruct((B * V, G * 3), means.dtype),
            jax.ShapeDtypeStruct((B * V, G), jnp.bool_),
        ],
        scratch_shapes=[
            pltpu.SemaphoreType.DMA((_NCHUNK,)),
            pltpu.SemaphoreType.DMA,
        ],
    )(f2, m2)

    return (
        f_out.reshape(B, V * G, C),
        m_out.reshape(B, V * G, 3),
        mask.reshape(B, V, G),
    )


# speedup vs baseline: 8.4855x; 8.4855x over previous
import jax
import jax.numpy as jnp
from jax.experimental import pallas as pl


def _copy_body(f_in, m_in, f_out, m_out, mask_out):
    f_out[...] = f_in[...]
    m_out[...] = m_in[...]
    mask_out[...] = jnp.ones(mask_out.shape, dtype=jnp.bool_)


def kernel(features, means, xy_coords, A):
    B, V, G, C = features.shape
    del xy_coords, A
    f2 = features.reshape(B * V * G, C)
    m2 = means.reshape(B * V, G * 3)

    ROWS = 4096
    n_prog = (B * V * G) // ROWS

    f_out, m_out, mask = pl.pallas_call(
        _copy_body,
        grid=(n_prog,),
        in_specs=[
            pl.BlockSpec((ROWS, C), lambda i: (i, 0)),
            pl.BlockSpec((B * V, G * 3), lambda i: (0, 0)),
        ],
        out_specs=[
            pl.BlockSpec((ROWS, C), lambda i: (i, 0)),
            pl.BlockSpec((B * V, G * 3), lambda i: (0, 0)),
            pl.BlockSpec((B * V, G), lambda i: (0, 0)),
        ],
        out_shape=[
            jax.ShapeDtypeStruct((B * V * G, C), features.dtype),
            jax.ShapeDtypeStruct((B * V, G * 3), means.dtype),
            jax.ShapeDtypeStruct((B * V, G), jnp.bool_),
        ],
    )(f2, m2)

    return (
        f_out.reshape(B, V * G, C),
        m_out.reshape(B, V * G, 3),
        mask.reshape(B, V, G),
    )
